# baseline (speedup 1.0000x reference)
import functools

import jax
import jax.numpy as jnp
from jax import lax
from jax.experimental import pallas as pl
from jax.experimental.pallas import tpu as pltpu
from jax.experimental.pallas import tpu_sc as plsc

N = 10000
EDG = 320000
DF = 128
DE = 16
H = 128
NLAYER = 4
NG = 64
GFD = 32

NC = 2
NS = 16
NW = NC * NS
EPT = EDG // NW
K = 80
NCH = EPT // K
NP = 10240
RPS = NP // NS

_BN_SCALE = 1.0 / (1.0 + 1e-5) ** 0.5


def _make_sc_agg():
    mesh = plsc.VectorSubcoreMesh(core_axis_name="c", subcore_axis_name="s")

    @functools.partial(
        pl.kernel,
        out_type=jax.ShapeDtypeStruct((NC, NP, H), jnp.float32),
        mesh=mesh,
        scratch_types=[
            pltpu.VMEM((1, K), jnp.int32),
            pltpu.VMEM((1, K), jnp.int32),
            pltpu.VMEM((1, K), jnp.int32),
            pltpu.VMEM((1, K), jnp.int32),
            pltpu.VMEM((K, H), jnp.float32),
            pltpu.VMEM((K, H), jnp.float32),
            pltpu.VMEM((K, H), jnp.float32),
            pltpu.VMEM_SHARED((NP, H), jnp.float32),
            pltpu.SemaphoreType.DMA,
            pltpu.SemaphoreType.DMA,
        ],
    )
    def sc_agg(h_hbm, e_hbm, src_hbm, dst_hbm, out_hbm,
               srcvA, srcvB, dstvA, dstvB, gathA, gathB, ebuf, aggsh,
               semA, semB):
        c = lax.axis_index("c")
        s = lax.axis_index("s")
        wid = c * NS + s

        def _zrow(r, _):
            for cc in range(H // 16):
                ebuf[r, pl.ds(cc * 16, 16)] = jnp.zeros((16,), jnp.float32)
            return 0
        lax.fori_loop(0, K, _zrow, 0)
        for j in range(RPS // K):
            pltpu.sync_copy(ebuf, aggsh.at[pl.ds(s * RPS + j * K, K)])
        plsc.subcore_barrier()

        def stage_and_fire(j, srcv, dstv, gath, sem):
            pltpu.sync_copy(src_hbm.at[wid, j], srcv)
            pltpu.sync_copy(dst_hbm.at[wid, j], dstv)
            pltpu.async_copy(h_hbm.at[srcv.at[0]], gath, sem)

        def process(j, srcv, dstv, gath, sem):
            pltpu.sync_copy(e_hbm.at[pl.ds(wid * EPT + j * K, K)], ebuf)
            pltpu.make_async_copy(h_hbm.at[srcv.at[0]], gath, sem).wait()

            def crow(r, _):
                for cc in range(H // 16):
                    sl = pl.ds(cc * 16, 16)
                    ebuf[r, sl] = jnp.maximum(gath[r, sl] + ebuf[r, sl], 0.0)
                return 0
            lax.fori_loop(0, K, crow, 0)
            pltpu.sync_copy(ebuf, aggsh.at[dstv.at[0]], add=True)

        stage_and_fire(0, srcvA, dstvA, gathA, semA)

        def pair(g, _):
            j0 = 2 * g
            stage_and_fire(j0 + 1, srcvB, dstvB, gathB, semB)
            process(j0, srcvA, dstvA, gathA, semA)
            stage_and_fire(j0 + 2, srcvA, dstvA, gathA, semA)
            process(j0 + 1, srcvB, dstvB, gathB, semB)
            return 0
        lax.fori_loop(0, (NCH - 1) // 2, pair, 0)
        process(NCH - 1, srcvA, dstvA, gathA, semA)
        plsc.subcore_barrier()

        for j in range(RPS // K):
            sl = pl.ds(s * RPS + j * K, K)
            pltpu.sync_copy(aggsh.at[sl], ebuf)
            pltpu.sync_copy(ebuf, out_hbm.at[c, sl])

    return sc_agg


_SC_AGG_CACHE = {}


def _sc_agg(h, e, src, dst):
    if "k" not in _SC_AGG_CACHE:
        _SC_AGG_CACHE["k"] = _make_sc_agg()
    return _SC_AGG_CACHE["k"](h, e, src, dst)


_RB = 2000
_NRB = N // _RB

_EB = 2000
_NEB = EDG // _EB


def _tc_in_body(x_ref, w_ref, b_ref, o_ref):
    o_ref[...] = (
        jnp.dot(x_ref[...], w_ref[...], preferred_element_type=jnp.float32, precision=lax.Precision.HIGHEST)
        + b_ref[...]
    )


def _tc_in(x, W_in, b_in):
    return pl.pallas_call(
        _tc_in_body,
        grid=(_NRB,),
        in_specs=[
            pl.BlockSpec((_RB, DF), lambda i: (i, 0)),
            pl.BlockSpec((DF, H), lambda i: (0, 0)),
            pl.BlockSpec((1, H), lambda i: (0, 0)),
        ],
        out_specs=pl.BlockSpec((_RB, H), lambda i: (i, 0)),
        out_shape=jax.ShapeDtypeStruct((N, H), jnp.float32),
    )(x, W_in, b_in.reshape(1, H))


def _tc_edges_body(ea_ref, we_ref, be_ref, o_ref):
    o_ref[...] = (
        jnp.dot(ea_ref[...], we_ref[0], preferred_element_type=jnp.float32, precision=lax.Precision.HIGHEST)
        + be_ref[0]
    )[None]


def _tc_edges(edge_attr, We, be):
    return pl.pallas_call(
        _tc_edges_body,
        grid=(NLAYER, _NEB),
        in_specs=[
            pl.BlockSpec((_EB, DE), lambda i, j: (j, 0)),
            pl.BlockSpec((1, DE, H), lambda i, j: (i, 0, 0)),
            pl.BlockSpec((1, 1, H), lambda i, j: (i, 0, 0)),
        ],
        out_specs=pl.BlockSpec((1, _EB, H), lambda i, j: (i, j, 0)),
        out_shape=jax.ShapeDtypeStruct((NLAYER, EDG, H), jnp.float32),
    )(edge_attr, We, be.reshape(NLAYER, 1, H))


def _tc_mlp_body(h_ref, pa_ref, pb_ref, w1_ref, b1_ref, w2_ref, b2_ref,
                 g_ref, bb_ref, o_ref):
    hb = h_ref[...]
    t = hb + pa_ref[0] + pb_ref[0]
    t = jnp.dot(t, w1_ref[...], preferred_element_type=jnp.float32, precision=lax.Precision.HIGHEST) + b1_ref[...]
    t = jnp.maximum(t, 0.0)
    t = jnp.dot(t, w2_ref[...], preferred_element_type=jnp.float32, precision=lax.Precision.HIGHEST) + b2_ref[...]
    t = t * (g_ref[...] * _BN_SCALE) + bb_ref[...]
    o_ref[...] = hb + jnp.maximum(t, 0.0)


def _tc_mlp(h, parts, W1, b1, W2, b2, bn_g, bn_b):
    vec = pl.BlockSpec((1, H), lambda i: (0, 0))
    mat = pl.BlockSpec((H, H), lambda i: (0, 0))
    blk = pl.BlockSpec((_RB, H), lambda i: (i, 0))
    pa = pl.BlockSpec((1, _RB, H), lambda i: (0, i, 0))
    pb = pl.BlockSpec((1, _RB, H), lambda i: (1, i, 0))
    return pl.pallas_call(
        _tc_mlp_body,
        grid=(_NRB,),
        in_specs=[blk, pa, pb, mat, vec, mat, vec, vec, vec],
        out_specs=blk,
        out_shape=jax.ShapeDtypeStruct((N, H), jnp.float32),
    )(h, parts, parts, W1, b1.reshape(1, H), W2, b2.reshape(1, H),
      bn_g.reshape(1, H), bn_b.reshape(1, H))


def _tc_pool_body(oh_ref, h_ref, g_ref):
    @pl.when(pl.program_id(0) == 0)
    def _():
        g_ref[...] = jnp.zeros_like(g_ref)
    g_ref[...] += jax.lax.dot_general(
        oh_ref[...], h_ref[...], (((0,), (0,)), ((), ())),
        preferred_element_type=jnp.float32, precision=lax.Precision.HIGHEST)


def _tc_pool(h, onehot):
    return pl.pallas_call(
        _tc_pool_body,
        grid=(_NRB,),
        in_specs=[
            pl.BlockSpec((_RB, NG), lambda i: (i, 0)),
            pl.BlockSpec((_RB, H), lambda i: (i, 0)),
        ],
        out_specs=pl.BlockSpec((NG, H), lambda i: (0, 0)),
        out_shape=jax.ShapeDtypeStruct((NG, H), jnp.float32),
    )(onehot, h)


def _softplus(x):
    return jnp.maximum(x, 0.0) + jnp.log1p(jnp.exp(-jnp.abs(x)))


def _tc_heads_body(g_ref, gf_ref, wp1_ref, bp1_ref, wp2_ref, bp2_ref,
                   wf1a_ref, wf1b_ref, bf1_ref, wf2_ref, bf2_ref,
                   wedl_ref, bedl_ref, edl_ref, z_ref):
    g = g_ref[...]
    z0 = jnp.maximum(
        jnp.dot(g, wp1_ref[...], preferred_element_type=jnp.float32, precision=lax.Precision.HIGHEST)
        + bp1_ref[...], 0.0)
    z0 = jnp.dot(z0, wp2_ref[...], preferred_element_type=jnp.float32, precision=lax.Precision.HIGHEST) + bp2_ref[...]
    nrm = jnp.sqrt(jnp.sum(z0 * z0, axis=1, keepdims=True))
    z_ref[...] = z0 / jnp.maximum(nrm, 1e-12)

    gc = (jnp.dot(g, wf1a_ref[...], preferred_element_type=jnp.float32, precision=lax.Precision.HIGHEST)
          + jnp.dot(gf_ref[...], wf1b_ref[...], preferred_element_type=jnp.float32, precision=lax.Precision.HIGHEST)
          + bf1_ref[...])
    gc = jnp.maximum(gc, 0.0)
    gc = jnp.dot(gc, wf2_ref[...], preferred_element_type=jnp.float32, precision=lax.Precision.HIGHEST) + bf2_ref[...]
    gc = jnp.maximum(gc, 0.0)
    o = jnp.dot(gc, wedl_ref[...], preferred_element_type=jnp.float32, precision=lax.Precision.HIGHEST) + bedl_ref[...]
    sp = _softplus(o)
    edl_ref[...] = jnp.concatenate(
        [o[:, 0:1], sp[:, 1:2] + 1e-6, sp[:, 2:3] + (1.0 + 1e-6),
         sp[:, 3:4] + 1e-6], axis=1)


def _tc_heads(g, global_feat, Wp1, bp1, Wp2, bp2, Wf1, bf1, Wf2, bf2,
              Wedl, bedl):
    full = lambda s: pl.BlockSpec(s, lambda: tuple(0 for _ in s))
    H2 = H // 2
    return pl.pallas_call(
        _tc_heads_body,
        in_specs=[
            full((NG, H)), full((NG, GFD)),
            full((H, H2)), full((1, H2)), full((H2, NG)), full((1, NG)),
            full((H, H)), full((GFD, H)), full((1, H)),
            full((H, H2)), full((1, H2)),
            full((H2, 4)), full((1, 4)),
        ],
        out_specs=[full((NG, 4)), full((NG, NG))],
        out_shape=[
            jax.ShapeDtypeStruct((NG, 4), jnp.float32),
            jax.ShapeDtypeStruct((NG, NG), jnp.float32),
        ],
    )(g, global_feat, Wp1, bp1.reshape(1, H2), Wp2, bp2.reshape(1, NG),
      Wf1[:H], Wf1[H:], bf1.reshape(1, H), Wf2, bf2.reshape(1, H2),
      Wedl, bedl.reshape(1, 4))


def kernel(x, edge_index, edge_attr, batch, global_feat, W_in, b_in, We, be,
           W1, b1, W2, b2, bn_g, bn_b, Wp1, bp1, Wp2, bp2, Wf1, bf1, Wf2,
           bf2, Wedl, bedl):
    src = edge_index[0].reshape(NW, NCH, 1, K)
    dst = edge_index[1].reshape(NW, NCH, 1, K)

    h = _tc_in(x, W_in, b_in)
    e_all = _tc_edges(edge_attr, We, be)

    for i in range(NLAYER):
        parts = _sc_agg(h, e_all[i], src, dst)
        h = _tc_mlp(h, parts, W1[i], b1[i], W2[i], b2[i], bn_g[i], bn_b[i])

    onehot = (batch[:, None] == jnp.arange(NG, dtype=jnp.int32)[None, :])
    g = _tc_pool(h, onehot.astype(jnp.float32))
    edl, z = _tc_heads(g, global_feat, Wp1, bp1, Wp2, bp2, Wf1, bf1, Wf2,
                       bf2, Wedl, bedl)
    return (edl, z, g)


# speedup vs baseline: 1.0730x; 1.0730x over previous
import functools

import jax
import jax.numpy as jnp
from jax import lax
from jax.experimental import pallas as pl
from jax.experimental.pallas import tpu as pltpu
from jax.experimental.pallas import tpu_sc as plsc

N = 10000
EDG = 320000
DF = 128
DE = 16
H = 128
NLAYER = 4
NG = 64
GFD = 32

NC = 2
NS = 16
NW = NC * NS
EPT = EDG // NW
K = 80
NCH = EPT // K
NP = 10240
RPS = NP // NS

_BN_SCALE = 1.0 / (1.0 + 1e-5) ** 0.5


def _make_sc_agg():
    mesh = plsc.VectorSubcoreMesh(core_axis_name="c", subcore_axis_name="s")

    @functools.partial(
        pl.kernel,
        out_type=jax.ShapeDtypeStruct((NC, NP, H), jnp.float32),
        mesh=mesh,
        scratch_types=[
            pltpu.VMEM((1, K), jnp.int32),
            pltpu.VMEM((1, K), jnp.int32),
            pltpu.VMEM((1, K), jnp.int32),
            pltpu.VMEM((1, K), jnp.int32),
            pltpu.VMEM((K, H), jnp.float32),
            pltpu.VMEM((K, H), jnp.float32),
            pltpu.VMEM((K, H), jnp.float32),
            pltpu.VMEM_SHARED((NP, H), jnp.float32),
            pltpu.SemaphoreType.DMA,
            pltpu.SemaphoreType.DMA,
        ],
    )
    def sc_agg(h_hbm, e_hbm, src_hbm, dst_hbm, out_hbm,
               srcvA, srcvB, dstvA, dstvB, gathA, gathB, ebuf, aggsh,
               semA, semB):
        c = lax.axis_index("c")
        s = lax.axis_index("s")
        wid = c * NS + s

        def _zrow(r, _):
            for cc in range(H // 16):
                ebuf[r, pl.ds(cc * 16, 16)] = jnp.zeros((16,), jnp.float32)
            return 0
        lax.fori_loop(0, K, _zrow, 0)
        for j in range(RPS // K):
            pltpu.sync_copy(ebuf, aggsh.at[pl.ds(s * RPS + j * K, K)])
        plsc.subcore_barrier()

        def stage_and_fire(j, srcv, dstv, gath, sem):
            pltpu.sync_copy(src_hbm.at[wid, j], srcv)
            pltpu.sync_copy(dst_hbm.at[wid, j], dstv)
            pltpu.async_copy(h_hbm.at[srcv.at[0]], gath, sem)

        def process(j, srcv, dstv, gath, sem):
            pltpu.sync_copy(e_hbm.at[pl.ds(wid * EPT + j * K, K)], ebuf)
            pltpu.make_async_copy(h_hbm.at[srcv.at[0]], gath, sem).wait()

            def crow(r, _):
                for cc in range(H // 16):
                    sl = pl.ds(cc * 16, 16)
                    ebuf[r, sl] = jnp.maximum(gath[r, sl] + ebuf[r, sl], 0.0)
                return 0
            lax.fori_loop(0, K, crow, 0)
            pltpu.sync_copy(ebuf, aggsh.at[dstv.at[0]], add=True)

        stage_and_fire(0, srcvA, dstvA, gathA, semA)

        def pair(g, _):
            j0 = 2 * g
            stage_and_fire(j0 + 1, srcvB, dstvB, gathB, semB)
            process(j0, srcvA, dstvA, gathA, semA)
            stage_and_fire(j0 + 2, srcvA, dstvA, gathA, semA)
            process(j0 + 1, srcvB, dstvB, gathB, semB)
            return 0
        lax.fori_loop(0, (NCH - 1) // 2, pair, 0)
        process(NCH - 1, srcvA, dstvA, gathA, semA)
        plsc.subcore_barrier()

        for j in range(RPS // K):
            sl = pl.ds(s * RPS + j * K, K)
            pltpu.sync_copy(aggsh.at[sl], ebuf)
            pltpu.sync_copy(ebuf, out_hbm.at[c, sl])

    return sc_agg


_SC_AGG_CACHE = {}


def _sc_agg(h, e, src, dst):
    if "k" not in _SC_AGG_CACHE:
        _SC_AGG_CACHE["k"] = _make_sc_agg()
    return _SC_AGG_CACHE["k"](h, e, src, dst)


_RB = 2000
_NRB = N // _RB

_EB = 2000
_NEB = EDG // _EB


def _tc_in_body(x_ref, w_ref, b_ref, o_ref):
    o_ref[...] = (
        jnp.dot(x_ref[...], w_ref[...], preferred_element_type=jnp.float32)
        + b_ref[...]
    )


def _tc_in(x, W_in, b_in):
    return pl.pallas_call(
        _tc_in_body,
        grid=(_NRB,),
        in_specs=[
            pl.BlockSpec((_RB, DF), lambda i: (i, 0)),
            pl.BlockSpec((DF, H), lambda i: (0, 0)),
            pl.BlockSpec((1, H), lambda i: (0, 0)),
        ],
        out_specs=pl.BlockSpec((_RB, H), lambda i: (i, 0)),
        out_shape=jax.ShapeDtypeStruct((N, H), jnp.float32),
    )(x, W_in, b_in.reshape(1, H))


def _tc_edges_body(ea_ref, we_ref, be_ref, o_ref):
    o_ref[...] = (
        jnp.dot(ea_ref[...], we_ref[0], preferred_element_type=jnp.float32)
        + be_ref[0]
    )[None]


def _tc_edges(edge_attr, We, be):
    return pl.pallas_call(
        _tc_edges_body,
        grid=(NLAYER, _NEB),
        in_specs=[
            pl.BlockSpec((_EB, DE), lambda i, j: (j, 0)),
            pl.BlockSpec((1, DE, H), lambda i, j: (i, 0, 0)),
            pl.BlockSpec((1, 1, H), lambda i, j: (i, 0, 0)),
        ],
        out_specs=pl.BlockSpec((1, _EB, H), lambda i, j: (i, j, 0)),
        out_shape=jax.ShapeDtypeStruct((NLAYER, EDG, H), jnp.float32),
    )(edge_attr, We, be.reshape(NLAYER, 1, H))


def _tc_mlp_body(h_ref, pa_ref, pb_ref, w1_ref, b1_ref, w2_ref, b2_ref,
                 g_ref, bb_ref, o_ref):
    hb = h_ref[...]
    t = hb + pa_ref[0] + pb_ref[0]
    t = jnp.dot(t, w1_ref[...], preferred_element_type=jnp.float32) + b1_ref[...]
    t = jnp.maximum(t, 0.0)
    t = jnp.dot(t, w2_ref[...], preferred_element_type=jnp.float32) + b2_ref[...]
    t = t * (g_ref[...] * _BN_SCALE) + bb_ref[...]
    o_ref[...] = hb + jnp.maximum(t, 0.0)


def _tc_mlp(h, parts, W1, b1, W2, b2, bn_g, bn_b):
    vec = pl.BlockSpec((1, H), lambda i: (0, 0))
    mat = pl.BlockSpec((H, H), lambda i: (0, 0))
    blk = pl.BlockSpec((_RB, H), lambda i: (i, 0))
    pa = pl.BlockSpec((1, _RB, H), lambda i: (0, i, 0))
    pb = pl.BlockSpec((1, _RB, H), lambda i: (1, i, 0))
    return pl.pallas_call(
        _tc_mlp_body,
        grid=(_NRB,),
        in_specs=[blk, pa, pb, mat, vec, mat, vec, vec, vec],
        out_specs=blk,
        out_shape=jax.ShapeDtypeStruct((N, H), jnp.float32),
    )(h, parts, parts, W1, b1.reshape(1, H), W2, b2.reshape(1, H),
      bn_g.reshape(1, H), bn_b.reshape(1, H))


def _tc_pool_body(oh_ref, h_ref, g_ref):
    @pl.when(pl.program_id(0) == 0)
    def _():
        g_ref[...] = jnp.zeros_like(g_ref)
    g_ref[...] += jax.lax.dot_general(
        oh_ref[...], h_ref[...], (((0,), (0,)), ((), ())),
        preferred_element_type=jnp.float32)


def _tc_pool(h, onehot):
    return pl.pallas_call(
        _tc_pool_body,
        grid=(_NRB,),
        in_specs=[
            pl.BlockSpec((_RB, NG), lambda i: (i, 0)),
            pl.BlockSpec((_RB, H), lambda i: (i, 0)),
        ],
        out_specs=pl.BlockSpec((NG, H), lambda i: (0, 0)),
        out_shape=jax.ShapeDtypeStruct((NG, H), jnp.float32),
    )(onehot, h)


def _softplus(x):
    return jnp.maximum(x, 0.0) + jnp.log1p(jnp.exp(-jnp.abs(x)))


def _tc_heads_body(g_ref, gf_ref, wp1_ref, bp1_ref, wp2_ref, bp2_ref,
                   wf1a_ref, wf1b_ref, bf1_ref, wf2_ref, bf2_ref,
                   wedl_ref, bedl_ref, edl_ref, z_ref):
    g = g_ref[...]
    z0 = jnp.maximum(
        jnp.dot(g, wp1_ref[...], preferred_element_type=jnp.float32)
        + bp1_ref[...], 0.0)
    z0 = jnp.dot(z0, wp2_ref[...], preferred_element_type=jnp.float32) + bp2_ref[...]
    nrm = jnp.sqrt(jnp.sum(z0 * z0, axis=1, keepdims=True))
    z_ref[...] = z0 / jnp.maximum(nrm, 1e-12)

    gc = (jnp.dot(g, wf1a_ref[...], preferred_element_type=jnp.float32)
          + jnp.dot(gf_ref[...], wf1b_ref[...], preferred_element_type=jnp.float32)
          + bf1_ref[...])
    gc = jnp.maximum(gc, 0.0)
    gc = jnp.dot(gc, wf2_ref[...], preferred_element_type=jnp.float32) + bf2_ref[...]
    gc = jnp.maximum(gc, 0.0)
    o = jnp.dot(gc, wedl_ref[...], preferred_element_type=jnp.float32) + bedl_ref[...]
    sp = _softplus(o)
    edl_ref[...] = jnp.concatenate(
        [o[:, 0:1], sp[:, 1:2] + 1e-6, sp[:, 2:3] + (1.0 + 1e-6),
         sp[:, 3:4] + 1e-6], axis=1)


def _tc_heads(g, global_feat, Wp1, bp1, Wp2, bp2, Wf1, bf1, Wf2, bf2,
              Wedl, bedl):
    full = lambda s: pl.BlockSpec(s, lambda: tuple(0 for _ in s))
    H2 = H // 2
    return pl.pallas_call(
        _tc_heads_body,
        in_specs=[
            full((NG, H)), full((NG, GFD)),
            full((H, H2)), full((1, H2)), full((H2, NG)), full((1, NG)),
            full((H, H)), full((GFD, H)), full((1, H)),
            full((H, H2)), full((1, H2)),
            full((H2, 4)), full((1, 4)),
        ],
        out_specs=[full((NG, 4)), full((NG, NG))],
        out_shape=[
            jax.ShapeDtypeStruct((NG, 4), jnp.float32),
            jax.ShapeDtypeStruct((NG, NG), jnp.float32),
        ],
    )(g, global_feat, Wp1, bp1.reshape(1, H2), Wp2, bp2.reshape(1, NG),
      Wf1[:H], Wf1[H:], bf1.reshape(1, H), Wf2, bf2.reshape(1, H2),
      Wedl, bedl.reshape(1, 4))


def kernel(x, edge_index, edge_attr, batch, global_feat, W_in, b_in, We, be,
           W1, b1, W2, b2, bn_g, bn_b, Wp1, bp1, Wp2, bp2, Wf1, bf1, Wf2,
           bf2, Wedl, bedl):
    src = edge_index[0].reshape(NW, NCH, 1, K)
    dst = edge_index[1].reshape(NW, NCH, 1, K)

    h = _tc_in(x, W_in, b_in)
    e_all = _tc_edges(edge_attr, We, be)

    for i in range(NLAYER):
        parts = _sc_agg(h, e_all[i], src, dst)
        h = _tc_mlp(h, parts, W1[i], b1[i], W2[i], b2[i], bn_g[i], bn_b[i])

    onehot = (batch[:, None] == jnp.arange(NG, dtype=jnp.int32)[None, :])
    g = _tc_pool(h, onehot.astype(jnp.float32))
    edl, z = _tc_heads(g, global_feat, Wp1, bp1, Wp2, bp2, Wf1, bf1, Wf2,
                       bf2, Wedl, bedl)
    return (edl, z, g)


# speedup vs baseline: 2.0652x; 1.9247x over previous
import functools

import jax
import jax.numpy as jnp
from jax import lax
from jax.experimental import pallas as pl
from jax.experimental.pallas import tpu as pltpu
from jax.experimental.pallas import tpu_sc as plsc

N = 10000
EDG = 320000
DF = 128
DE = 16
H = 128
HH = H // 2
NLAYER = 4
NG = 64
GFD = 32

NC = 2
NS = 16
NW = NC * NS
EPT = EDG // NW
K = 80
NCH = EPT // K
NP = 10240
RPS = NP // NS

_BN_SCALE = 1.0 / (1.0 + 1e-5) ** 0.5


def _make_sc_agg():
    mesh = plsc.VectorSubcoreMesh(core_axis_name="c", subcore_axis_name="s")

    @functools.partial(
        pl.kernel,
        out_type=jax.ShapeDtypeStruct((NC, NP, H), jnp.float32),
        mesh=mesh,
        scratch_types=[
            pltpu.VMEM((1, K), jnp.int32),
            pltpu.VMEM((1, K), jnp.int32),
            pltpu.VMEM((1, K), jnp.int32),
            pltpu.VMEM((1, K), jnp.int32),
            pltpu.VMEM((K, H), jnp.float32),
            pltpu.VMEM((K, H), jnp.float32),
            pltpu.VMEM((K, H), jnp.float32),
            pltpu.VMEM_SHARED((NP, H), jnp.float32),
            pltpu.SemaphoreType.DMA,
            pltpu.SemaphoreType.DMA,
            pltpu.SemaphoreType.DMA,
            pltpu.SemaphoreType.DMA,
            pltpu.SemaphoreType.DMA,
        ],
    )
    def sc_agg(h_hbm, e_hbm, src_hbm, dst_hbm, out_hbm,
               srcvA, srcvB, dstvA, dstvB, gathA, gathB, ebuf,
               aggsh, semGA, semGB, semE, semSA, semSB):
        c = lax.axis_index("c")
        s = lax.axis_index("s")
        wid = c * NS + s

        def _zrow(r, _):
            for cc in range(H // 16):
                ebuf[r, pl.ds(cc * 16, 16)] = jnp.zeros((16,), jnp.float32)
            return 0
        lax.fori_loop(0, K, _zrow, 0)
        for j in range(RPS // K):
            pltpu.sync_copy(ebuf, aggsh.at[pl.ds(s * RPS + j * K, K)])
        plsc.subcore_barrier()

        def fire_e(j):
            pltpu.async_copy(
                e_hbm.at[pl.ds(wid * EPT + j * K, K)], ebuf, semE)

        def stage(j, srcv, dstv, gath, semG, semS, drain):
            if drain:
                pltpu.make_async_copy(
                    gath, aggsh.at[dstv.at[0]], semS).wait()
            pltpu.sync_copy(src_hbm.at[wid, j], srcv)
            pltpu.sync_copy(dst_hbm.at[wid, j], dstv)
            pltpu.async_copy(h_hbm.at[srcv.at[0]], gath, semG)

        def process(j, srcv, dstv, gath, semG, semS, fire_next):
            pltpu.make_async_copy(h_hbm.at[srcv.at[0]], gath, semG).wait()
            pltpu.make_async_copy(
                e_hbm.at[pl.ds(wid * EPT + j * K, K)], ebuf, semE).wait()

            def crow(r, _):
                for cc in range(H // 16):
                    sl = pl.ds(cc * 16, 16)
                    gath[r, sl] = jnp.maximum(gath[r, sl] + ebuf[r, sl], 0.0)
                return 0
            lax.fori_loop(0, K, crow, 0)
            if fire_next:
                fire_e(j + 1)
            pltpu.async_copy(gath, aggsh.at[dstv.at[0]], semS, add=True)

        A = (srcvA, dstvA, gathA, semGA, semSA)
        B = (srcvB, dstvB, gathB, semGB, semSB)

        stage(0, *A, drain=False)
        fire_e(0)
        stage(1, *B, drain=False)

        def pair(g, _):
            j0 = 2 * g
            process(j0, *A, fire_next=True)
            stage(j0 + 2, *A, drain=True)
            process(j0 + 1, *B, fire_next=True)
            stage(j0 + 3, *B, drain=True)
            return 0
        lax.fori_loop(0, (NCH - 3) // 2, pair, 0)
        process(NCH - 3, *A, fire_next=True)
        stage(NCH - 1, *A, drain=True)
        process(NCH - 2, *B, fire_next=True)
        process(NCH - 1, *A, fire_next=False)
        pltpu.make_async_copy(gathA, aggsh.at[dstvA.at[0]], semSA).wait()
        pltpu.make_async_copy(gathB, aggsh.at[dstvB.at[0]], semSB).wait()
        plsc.subcore_barrier()

        for j in range(RPS // K):
            sl = pl.ds(s * RPS + j * K, K)
            pltpu.sync_copy(aggsh.at[sl], gathA)
            pltpu.sync_copy(gathA, out_hbm.at[c, sl])

    return sc_agg


_SC_AGG_CACHE = {}


def _sc_agg(h2, e2, src, dst):
    if "k" not in _SC_AGG_CACHE:
        _SC_AGG_CACHE["k"] = _make_sc_agg()
    return _SC_AGG_CACHE["k"](h2, e2, src, dst)


_RB = 2000
_NRB = N // _RB

_EB = 8000
_NEB = EDG // _EB


def _tc_in_body(x_ref, w_ref, b_ref, o_ref):
    o_ref[...] = (
        jnp.dot(x_ref[...], w_ref[...], preferred_element_type=jnp.float32)
        + b_ref[...]
    )


def _tc_in(x, W_in, b_in):
    return pl.pallas_call(
        _tc_in_body,
        grid=(_NRB,),
        in_specs=[
            pl.BlockSpec((_RB, DF), lambda i: (i, 0)),
            pl.BlockSpec((DF, H), lambda i: (0, 0)),
            pl.BlockSpec((1, H), lambda i: (0, 0)),
        ],
        out_specs=pl.BlockSpec((_RB, H), lambda i: (i, 0)),
        out_shape=jax.ShapeDtypeStruct((N, H), jnp.float32),
    )(x, W_in, b_in.reshape(1, H))


def _tc_edges_body(ea_ref, we_ref, be_ref, o_ref):
    o_ref[...] = (
        jnp.dot(ea_ref[...], we_ref[...], preferred_element_type=jnp.float32)
        + be_ref[...]
    )


def _tc_edges(edge_attr, We_i, be_i):
    return pl.pallas_call(
        _tc_edges_body,
        grid=(_NEB,),
        in_specs=[
            pl.BlockSpec((_EB, DE), lambda j: (j, 0)),
            pl.BlockSpec((DE, H), lambda j: (0, 0)),
            pl.BlockSpec((1, H), lambda j: (0, 0)),
        ],
        out_specs=pl.BlockSpec((_EB, H), lambda j: (j, 0)),
        out_shape=jax.ShapeDtypeStruct((EDG, H), jnp.float32),
    )(edge_attr, We_i, be_i.reshape(1, H))


def _tc_mlp_body(h_ref, pa_ref, pb_ref, w1_ref, b1_ref, w2_ref, b2_ref,
                 g_ref, bb_ref, o_ref):
    hb = h_ref[...]
    t = hb + (pa_ref[0] + pb_ref[0])
    t = jnp.dot(t, w1_ref[...], preferred_element_type=jnp.float32) + b1_ref[...]
    t = jnp.maximum(t, 0.0)
    t = jnp.dot(t, w2_ref[...], preferred_element_type=jnp.float32) + b2_ref[...]
    t = t * (g_ref[...] * _BN_SCALE) + bb_ref[...]
    o_ref[...] = hb + jnp.maximum(t, 0.0)


def _tc_mlp(h, parts, W1, b1, W2, b2, bn_g, bn_b):
    vec = pl.BlockSpec((1, H), lambda i: (0, 0))
    mat = pl.BlockSpec((H, H), lambda i: (0, 0))
    blk = pl.BlockSpec((_RB, H), lambda i: (i, 0))
    pa = pl.BlockSpec((1, _RB, H), lambda i: (0, i, 0))
    pb = pl.BlockSpec((1, _RB, H), lambda i: (1, i, 0))
    return pl.pallas_call(
        _tc_mlp_body,
        grid=(_NRB,),
        in_specs=[blk, pa, pb, mat, vec, mat, vec, vec, vec],
        out_specs=blk,
        out_shape=jax.ShapeDtypeStruct((N, H), jnp.float32),
    )(h, parts, parts, W1, b1.reshape(1, H), W2, b2.reshape(1, H),
      bn_g.reshape(1, H), bn_b.reshape(1, H))


def _tc_pool_body(oh_ref, h_ref, g_ref):
    @pl.when(pl.program_id(0) == 0)
    def _():
        g_ref[...] = jnp.zeros_like(g_ref)
    g_ref[...] += jax.lax.dot_general(
        oh_ref[...], h_ref[...], (((0,), (0,)), ((), ())),
        preferred_element_type=jnp.float32)


def _tc_pool(h, onehot):
    return pl.pallas_call(
        _tc_pool_body,
        grid=(_NRB,),
        in_specs=[
            pl.BlockSpec((_RB, NG), lambda i: (i, 0)),
            pl.BlockSpec((_RB, H), lambda i: (i, 0)),
        ],
        out_specs=pl.BlockSpec((NG, H), lambda i: (0, 0)),
        out_shape=jax.ShapeDtypeStruct((NG, H), jnp.float32),
    )(onehot, h)


def _softplus(x):
    return jnp.maximum(x, 0.0) + jnp.log1p(jnp.exp(-jnp.abs(x)))


def _tc_heads_body(g_ref, gf_ref, wp1_ref, bp1_ref, wp2_ref, bp2_ref,
                   wf1a_ref, wf1b_ref, bf1_ref, wf2_ref, bf2_ref,
                   wedl_ref, bedl_ref, edl_ref, z_ref):
    g = g_ref[...]
    z0 = jnp.maximum(
        jnp.dot(g, wp1_ref[...], preferred_element_type=jnp.float32)
        + bp1_ref[...], 0.0)
    z0 = jnp.dot(z0, wp2_ref[...], preferred_element_type=jnp.float32) + bp2_ref[...]
    nrm = jnp.sqrt(jnp.sum(z0 * z0, axis=1, keepdims=True))
    z_ref[...] = z0 / jnp.maximum(nrm, 1e-12)

    gc = (jnp.dot(g, wf1a_ref[...], preferred_element_type=jnp.float32)
          + jnp.dot(gf_ref[...], wf1b_ref[...], preferred_element_type=jnp.float32)
          + bf1_ref[...])
    gc = jnp.maximum(gc, 0.0)
    gc = jnp.dot(gc, wf2_ref[...], preferred_element_type=jnp.float32) + bf2_ref[...]
    gc = jnp.maximum(gc, 0.0)
    o = jnp.dot(gc, wedl_ref[...], preferred_element_type=jnp.float32) + bedl_ref[...]
    sp = _softplus(o)
    edl_ref[...] = jnp.concatenate(
        [o[:, 0:1], sp[:, 1:2] + 1e-6, sp[:, 2:3] + (1.0 + 1e-6),
         sp[:, 3:4] + 1e-6], axis=1)


def _tc_heads(g, global_feat, Wp1, bp1, Wp2, bp2, Wf1, bf1, Wf2, bf2,
              Wedl, bedl):
    full = lambda s: pl.BlockSpec(s, lambda: tuple(0 for _ in s))
    H2 = H // 2
    return pl.pallas_call(
        _tc_heads_body,
        in_specs=[
            full((NG, H)), full((NG, GFD)),
            full((H, H2)), full((1, H2)), full((H2, NG)), full((1, NG)),
            full((H, H)), full((GFD, H)), full((1, H)),
            full((H, H2)), full((1, H2)),
            full((H2, 4)), full((1, 4)),
        ],
        out_specs=[full((NG, 4)), full((NG, NG))],
        out_shape=[
            jax.ShapeDtypeStruct((NG, 4), jnp.float32),
            jax.ShapeDtypeStruct((NG, NG), jnp.float32),
        ],
    )(g, global_feat, Wp1, bp1.reshape(1, H2), Wp2, bp2.reshape(1, NG),
      Wf1[:H], Wf1[H:], bf1.reshape(1, H), Wf2, bf2.reshape(1, H2),
      Wedl, bedl.reshape(1, 4))


def kernel(x, edge_index, edge_attr, batch, global_feat, W_in, b_in, We, be,
           W1, b1, W2, b2, bn_g, bn_b, Wp1, bp1, Wp2, bp2, Wf1, bf1, Wf2,
           bf2, Wedl, bedl):
    src = edge_index[0].reshape(NW, NCH, 1, K)
    dst = edge_index[1].reshape(NW, NCH, 1, K)

    h = _tc_in(x, W_in, b_in)

    for i in range(NLAYER):
        e = _tc_edges(edge_attr, We[i], be[i])
        parts = _sc_agg(h, e, src, dst)
        h = _tc_mlp(h, parts, W1[i], b1[i], W2[i], b2[i], bn_g[i], bn_b[i])

    onehot = (batch[:, None] == jnp.arange(NG, dtype=jnp.int32)[None, :])
    g = _tc_pool(h, onehot.astype(jnp.float32))
    edl, z = _tc_heads(g, global_feat, Wp1, bp1, Wp2, bp2, Wf1, bf1, Wf2,
                       bf2, Wedl, bedl)
    return (edl, z, g)
